# Initial kernel scaffold; baseline (speedup 1.0000x reference)
import jax
import jax.numpy as jnp
from jax.experimental import pallas as pl


def kernel(x, edge_index, params):
    raise NotImplementedError("write your pallas kernel here")


# speedup vs baseline: 1.0488x; 1.0488x over previous
import functools
import jax
import jax.numpy as jnp
from jax.experimental import pallas as pl
from jax.experimental.pallas import tpu as pltpu

N = 10000
E = 320000
F_IN = 128
EMB = 512
HID = 768
HEADS = 4
C = HID // HEADS

NP = 10240
BN_ROWS = 512

_INV_SQRT = 1.0 / (1.0 + 1e-5) ** 0.5


def _mm_kernel(x_ref, w_ref, s_ref, o_ref, *, act):
    acc = jnp.dot(x_ref[...], w_ref[...], preferred_element_type=jnp.float32)
    y = acc * s_ref[0:1, :] + s_ref[1:2, :]
    if act:
        y = jnp.where(y > 0, y, 0.01 * y)
    o_ref[...] = y


def _dense(x, w, scale_shift, act):
    K = x.shape[1]
    M = w.shape[1]
    grid = (NP // BN_ROWS,)
    return pl.pallas_call(
        functools.partial(_mm_kernel, act=act),
        grid=grid,
        in_specs=[
            pl.BlockSpec((BN_ROWS, K), lambda i: (i, 0)),
            pl.BlockSpec((K, M), lambda i: (0, 0)),
            pl.BlockSpec((2, M), lambda i: (0, 0)),
        ],
        out_specs=pl.BlockSpec((BN_ROWS, M), lambda i: (i, 0)),
        out_shape=jax.ShapeDtypeStruct((NP, M), jnp.float32),
    )(x, w, scale_shift)


def _fuse_bn(b, gamma, beta):
    scale = gamma * _INV_SQRT
    shift = scale * b + beta
    return jnp.stack([scale, shift])


def kernel(x, edge_index, params):
    p = params
    src = edge_index[0]
    dst = edge_index[1]

    xp = jnp.zeros((NP, F_IN), jnp.float32).at[:N].set(x)

    h = _dense(xp, p['W1'], _fuse_bn(p['b1'], p['g1'], p['be1']), True)
    h = _dense(h, p['W2'], _fuse_bn(p['b2'], p['g2'], p['be2']), True)

    for layer in (1, 2):
        W = p[f'Wg{layer}']
        a_s = p[f'as{layer}']
        a_d = p[f'ad{layer}']
        bias = p[f'bg{layer}']
        gb = p[f'gb{layer}']
        bb = p[f'bb{layer}']

        ident = jnp.stack([jnp.ones((HID,), jnp.float32),
                           jnp.zeros((HID,), jnp.float32)])
        hp = _dense(h, W, ident, False)

        A = jnp.zeros((HID, 2 * HEADS), jnp.float32)
        for hh in range(HEADS):
            A = A.at[hh * C:(hh + 1) * C, hh].set(a_s[hh])
            A = A.at[hh * C:(hh + 1) * C, HEADS + hh].set(a_d[hh])
        aa_ss = jnp.stack([jnp.ones((2 * HEADS,), jnp.float32),
                           jnp.zeros((2 * HEADS,), jnp.float32)])
        aa = _dense(hp, A, aa_ss, False)
        als = aa[:N, :HEADS]
        ald = aa[:N, HEADS:]

        K_h = jnp.maximum(jnp.max(als, axis=0) + jnp.max(ald, axis=0), 0.0)

        e = als[src] + ald[dst]
        e = jnp.where(e > 0, e, 0.2 * e)
        w = jnp.exp(e - K_h[None, :])
        denom = jax.ops.segment_sum(w, dst, num_segments=N)

        msg = hp[:N].reshape(N, HEADS, C)[src] * w[:, :, None]
        agg = jax.ops.segment_sum(msg, dst, num_segments=N)
        agg = agg / (denom[:, :, None] + 1e-16)
        out = agg.reshape(N, HID) + bias

        h = jnp.zeros((NP, HID), jnp.float32).at[:N].set(out)
        scale = gb * _INV_SQRT
        h = h * scale[None, :] + (jnp.zeros((NP, HID), jnp.float32)
                                  .at[:N].set(jnp.broadcast_to(bb, (N, HID))))
        h = jnp.where(h > 0, h, 0.01 * h)

    h = _dense(h, p['L1W'], _fuse_bn(p['L1b'], jnp.ones((HID,)), jnp.zeros((HID,))), True)
    h = _dense(h, p['L2W'], _fuse_bn(p['L2b'], jnp.ones((HID // 2,)), jnp.zeros((HID // 2,))), True)
    L3Wp = jnp.zeros((HID // 2, 128), jnp.float32).at[:, :1].set(p['L3W'])
    ss = jnp.stack([jnp.ones((128,), jnp.float32),
                    jnp.zeros((128,), jnp.float32).at[0].set(p['L3b'][0])])
    out = _dense(h, L3Wp, ss, False)
    return out[:N, :1]


# speedup vs baseline: 7.3201x; 6.9798x over previous
import functools
import jax
import jax.numpy as jnp
from jax import lax
from jax.experimental import pallas as pl
from jax.experimental.pallas import tpu as pltpu
from jax.experimental.pallas import tpu_sc as plsc

N = 10000
E = 320000
F_IN = 128
EMB = 512
HID = 768
HEADS = 4
C = HID // HEADS

NP = 10240
BN_ROWS = 512

NSUB = 16
NW = 32
R = 80
NRNG = NP // R
PHASES = NRNG // NW
RCAP = 2944
G = 16
NBLK = RCAP // G
NE = NRNG * RCAP
ECH = 512
NCH = NE // NW // ECH

_INV_SQRT = 1.0 / (1.0 + 1e-5) ** 0.5


def _mm_kernel(x_ref, w_ref, s_ref, o_ref, *, act):
    acc = jnp.dot(x_ref[...], w_ref[...], preferred_element_type=jnp.float32)
    y = acc * s_ref[0:1, :] + s_ref[1:2, :]
    if act:
        y = jnp.where(y > 0, y, 0.01 * y)
    o_ref[...] = y


def _dense(x, w, scale_shift, act):
    K = x.shape[1]
    M = w.shape[1]
    return pl.pallas_call(
        functools.partial(_mm_kernel, act=act),
        grid=(NP // BN_ROWS,),
        in_specs=[
            pl.BlockSpec((BN_ROWS, K), lambda i: (i, 0)),
            pl.BlockSpec((K, M), lambda i: (0, 0)),
            pl.BlockSpec((2, M), lambda i: (0, 0)),
        ],
        out_specs=pl.BlockSpec((BN_ROWS, M), lambda i: (i, 0)),
        out_shape=jax.ShapeDtypeStruct((NP, M), jnp.float32),
    )(x, w, scale_shift)


def _fuse_bn(b, gamma, beta):
    scale = gamma * _INV_SQRT
    shift = scale * b + beta
    return jnp.stack([scale, shift])


def _full16(v):
    return jnp.full((16,), v, jnp.int32)


_MESH = dict(core_axis_name="c", subcore_axis_name="s")


def _w_body(aaf, srcf, dstgf, kvec, wout,
            aa_v, src_v, dstg_v, wbuf_v, kv_v):
    c = lax.axis_index("c")
    s = lax.axis_index("s")
    wid = c * NSUB + s

    pltpu.sync_copy(kvec, kv_v)
    pltpu.sync_copy(aaf, aa_v)
    lanes0 = lax.iota(jnp.int32, 16)

    def chunk_body(ch, carry):
        offe = wid * (NCH * ECH) + ch * ECH
        pltpu.sync_copy(srcf.at[pl.ds(offe, ECH)], src_v)
        pltpu.sync_copy(dstgf.at[pl.ds(offe, ECH)], dstg_v)

        def grp_body(q, qcarry):
            src16 = src_v[pl.ds(q * 16, 16)]
            dstg16 = dstg_v[pl.ds(q * 16, 16)]
            ssl = src16 * 8
            dsl = dstg16 * 8
            for h in range(HEADS):
                als = plsc.load_gather(aa_v, [ssl + h])
                ald = plsc.load_gather(aa_v, [dsl + (HEADS + h)])
                e = als + ald
                e = jnp.where(e > 0, e, e * 0.2)
                kh = plsc.load_gather(kv_v, [_full16(h)])
                w16 = jnp.exp(e - kh)
                plsc.store_scatter(
                    wbuf_v, [(q * 16 + lanes0) * HEADS + h], w16)
            return qcarry

        lax.fori_loop(0, ECH // 16, grp_body, 0)
        pltpu.sync_copy(wbuf_v, wout.at[pl.ds(offe * HEADS, ECH * HEADS)])
        return carry

    lax.fori_loop(0, NCH, chunk_body, 0)


def _w_phase(aa, srcf, dstgf, kvec):
    f = pl.kernel(
        _w_body,
        out_type=[jax.ShapeDtypeStruct((NE * HEADS,), jnp.float32)],
        mesh=plsc.VectorSubcoreMesh(num_cores=2, num_subcores=NSUB, **_MESH),
        compiler_params=pltpu.CompilerParams(needs_layout_passes=False),
        scratch_types=[
            pltpu.VMEM((NP * 8,), jnp.float32),
            pltpu.VMEM((ECH,), jnp.int32),
            pltpu.VMEM((ECH,), jnp.int32),
            pltpu.VMEM((ECH * HEADS,), jnp.float32),
            pltpu.VMEM((128,), jnp.float32),
        ],
    )
    (w,) = f(aa.reshape(NP * 8), srcf, dstgf, kvec)
    return w


def _agg_body(hp, wall, srcs3, dstl2, zacc, zden,
              agg, denp,
              srcs_v, dstl_v, wrng_v, rows_v, acc_v, den_v, sem):
    c = lax.axis_index("c")
    s = lax.axis_index("s")
    wid = c * NSUB + s
    lanes0 = lax.iota(jnp.int32, 16)

    def range_body(p, carry):
        rid = p * NW + wid
        pltpu.sync_copy(srcs3.at[rid], srcs_v)
        pltpu.sync_copy(dstl2.at[rid], dstl_v)
        pltpu.sync_copy(wall.at[rid], wrng_v)
        pltpu.sync_copy(zacc, acc_v)
        pltpu.sync_copy(zden, den_v)

        def blk_body(b, bcarry):
            pltpu.async_copy(hp.at[srcs_v.at[b]], rows_v, sem).wait()

            dst16 = dstl_v[pl.ds(b * 16, 16)]
            widx = (b * 16 + lanes0) * HEADS
            for h in range(HEADS):
                wv = plsc.load_gather(wrng_v, [widx + h])
                plsc.addupdate_scatter(den_v, [dst16 * HEADS + h], wv)

            def edge_scale(i, ecarry):
                dstb = plsc.load_gather(dstl_v, [_full16(b * 16 + i)])
                rowbase = dstb * HID
                for h in range(HEADS):
                    wb = plsc.load_gather(
                        wrng_v, [_full16((b * 16 + i) * HEADS + h)])
                    for j in range(C // 16):
                        col = h * C + j * 16
                        seg = rows_v[i, pl.ds(col, 16)]
                        plsc.addupdate_scatter(
                            acc_v, [rowbase + col + lanes0], seg * wb)
                return ecarry

            lax.fori_loop(0, G, edge_scale, 0)
            return bcarry

        lax.fori_loop(0, NBLK, blk_body, 0)

        pltpu.sync_copy(acc_v.at[pl.ds(0, R * HID)],
                        agg.at[pl.ds(rid * R * HID, R * HID)])
        pltpu.sync_copy(den_v.at[pl.ds(0, 384)], denp.at[rid])
        return carry

    lax.fori_loop(0, PHASES, range_body, 0)


def _agg_phase(hp, wall, srcs3, dstl2, zacc, zden):
    f = pl.kernel(
        _agg_body,
        out_type=[
            jax.ShapeDtypeStruct((NP * HID,), jnp.float32),
            jax.ShapeDtypeStruct((NRNG, 384), jnp.float32),
        ],
        mesh=plsc.VectorSubcoreMesh(num_cores=2, num_subcores=NSUB, **_MESH),
        compiler_params=pltpu.CompilerParams(needs_layout_passes=False),
        scratch_types=[
            pltpu.VMEM((NBLK, G), jnp.int32),
            pltpu.VMEM((RCAP,), jnp.int32),
            pltpu.VMEM((RCAP * HEADS,), jnp.float32),
            pltpu.VMEM((G, HID), jnp.float32),
            pltpu.VMEM(((R + 1) * HID,), jnp.float32),
            pltpu.VMEM((384,), jnp.float32),
            pltpu.SemaphoreType.DMA,
        ],
    )
    return f(hp, wall.reshape(NRNG, RCAP * HEADS), srcs3, dstl2, zacc, zden)


def _prep_edges(src, dst):
    order = jnp.argsort(dst)
    src_s = src[order]
    dst_s = dst[order]
    bounds = jnp.searchsorted(dst_s, jnp.arange(NRNG + 1) * R).astype(jnp.int32)
    src_pad = jnp.concatenate([src_s, jnp.zeros((RCAP,), jnp.int32)])
    dst_pad = jnp.concatenate([dst_s, jnp.zeros((RCAP,), jnp.int32)])
    ar = jnp.arange(RCAP, dtype=jnp.int32)
    idx = bounds[:NRNG, None] + ar[None, :]
    valid = idx < bounds[1:, None]
    srcs = jnp.where(valid, src_pad[idx], 0)
    dstg = jnp.where(valid, dst_pad[idx], 0)
    rid = jnp.arange(NRNG, dtype=jnp.int32)
    dstl = jnp.where(valid, dst_pad[idx] - rid[:, None] * R, R)
    srcs3 = srcs.reshape(NRNG, NBLK, G)
    srcf = srcs.reshape(NE)
    dstgf = dstg.reshape(NE)
    dstl2 = dstl.reshape(NRNG, RCAP)
    return srcs3, srcf, dstgf, dstl2


def kernel(x, edge_index, params):
    p = params
    src = edge_index[0]
    dst = edge_index[1]

    srcs3, srcf, dstgf, dstl2 = _prep_edges(src, dst)
    zacc = jnp.zeros(((R + 1) * HID,), jnp.float32)
    zden = jnp.zeros((384,), jnp.float32)

    xp = jnp.zeros((NP, F_IN), jnp.float32).at[:N].set(x)

    h = _dense(xp, p['W1'], _fuse_bn(p['b1'], p['g1'], p['be1']), True)
    h = _dense(h, p['W2'], _fuse_bn(p['b2'], p['g2'], p['be2']), True)

    for layer in (1, 2):
        W = p[f'Wg{layer}']
        a_s = p[f'as{layer}']
        a_d = p[f'ad{layer}']
        bias = p[f'bg{layer}']
        gb = p[f'gb{layer}']
        bb = p[f'bb{layer}']

        As = jnp.zeros((HID, HEADS), jnp.float32)
        Ad = jnp.zeros((HID, HEADS), jnp.float32)
        for hh in range(HEADS):
            As = As.at[hh * C:(hh + 1) * C, hh].set(a_s[hh])
            Ad = Ad.at[hh * C:(hh + 1) * C, hh].set(a_d[hh])
        Waa = jnp.concatenate([jnp.dot(W, As), jnp.dot(W, Ad)], axis=1)

        ident = jnp.stack([jnp.ones((HID,), jnp.float32),
                           jnp.zeros((HID,), jnp.float32)])
        hp = _dense(h, W, ident, False)
        ident8 = jnp.stack([jnp.ones((8,), jnp.float32),
                            jnp.zeros((8,), jnp.float32)])
        aa = _dense(h, Waa, ident8, False)

        als = aa[:N, :HEADS]
        ald = aa[:N, HEADS:]
        K_h = jnp.maximum(jnp.max(als, axis=0) + jnp.max(ald, axis=0), 0.0)
        kvec = jnp.zeros((128,), jnp.float32).at[:HEADS].set(K_h)

        wall = _w_phase(aa, srcf, dstgf, kvec)
        aggf, denp = _agg_phase(hp, wall, srcs3, dstl2, zacc, zden)

        agg = aggf.reshape(NP, HID)[:N]
        den = denp[:, :R * HEADS].reshape(NP, HEADS)[:N]
        aggn = agg.reshape(N, HEADS, C) / (den[:, :, None] + 1e-16)
        out = aggn.reshape(N, HID) + bias

        scale = gb * _INV_SQRT
        y = out * scale[None, :] + bb[None, :]
        y = jnp.where(y > 0, y, 0.01 * y)
        h = jnp.zeros((NP, HID), jnp.float32).at[:N].set(y)

    h = _dense(h, p['L1W'], _fuse_bn(p['L1b'], jnp.ones((HID,)), jnp.zeros((HID,))), True)
    h = _dense(h, p['L2W'], _fuse_bn(p['L2b'], jnp.ones((HID // 2,)), jnp.zeros((HID // 2,))), True)
    L3Wp = jnp.zeros((HID // 2, 128), jnp.float32).at[:, :1].set(p['L3W'])
    ss = jnp.stack([jnp.ones((128,), jnp.float32),
                    jnp.zeros((128,), jnp.float32).at[0].set(p['L3b'][0])])
    out = _dense(h, L3Wp, ss, False)
    return out[:N, :1]


# speedup vs baseline: 7.5511x; 1.0316x over previous
import functools
import jax
import jax.numpy as jnp
from jax import lax
from jax.experimental import pallas as pl
from jax.experimental.pallas import tpu as pltpu
from jax.experimental.pallas import tpu_sc as plsc

N = 10000
E = 320000
F_IN = 128
EMB = 512
HID = 768
HEADS = 4
C = HID // HEADS

NP = 10240
BN_ROWS = 512

NSUB = 16
NW = 32
R = 80
NRNG = NP // R
PHASES = NRNG // NW
RCAP = 2944
G = 32
NBLK = RCAP // G
NE = NRNG * RCAP
ECH = 512
NCH = NE // NW // ECH

_INV_SQRT = 1.0 / (1.0 + 1e-5) ** 0.5


def _mm_kernel(x_ref, w_ref, s_ref, o_ref, *, act):
    acc = jnp.dot(x_ref[...], w_ref[...], preferred_element_type=jnp.float32)
    y = acc * s_ref[0:1, :] + s_ref[1:2, :]
    if act:
        y = jnp.where(y > 0, y, 0.01 * y)
    o_ref[...] = y


def _dense(x, w, scale_shift, act):
    K = x.shape[1]
    M = w.shape[1]
    return pl.pallas_call(
        functools.partial(_mm_kernel, act=act),
        grid=(NP // BN_ROWS,),
        in_specs=[
            pl.BlockSpec((BN_ROWS, K), lambda i: (i, 0)),
            pl.BlockSpec((K, M), lambda i: (0, 0)),
            pl.BlockSpec((2, M), lambda i: (0, 0)),
        ],
        out_specs=pl.BlockSpec((BN_ROWS, M), lambda i: (i, 0)),
        out_shape=jax.ShapeDtypeStruct((NP, M), jnp.float32),
    )(x, w, scale_shift)


def _fuse_bn(b, gamma, beta):
    scale = gamma * _INV_SQRT
    shift = scale * b + beta
    return jnp.stack([scale, shift])


def _full16(v):
    return jnp.full((16,), v, jnp.int32)


_MESH = dict(core_axis_name="c", subcore_axis_name="s")


def _w_body(aaf, srcf, dstgf, kvec, wout,
            aa_v, src_v, dstg_v, wbuf_v, kv_v):
    c = lax.axis_index("c")
    s = lax.axis_index("s")
    wid = c * NSUB + s

    pltpu.sync_copy(kvec, kv_v)
    pltpu.sync_copy(aaf, aa_v)
    lanes0 = lax.iota(jnp.int32, 16)

    def chunk_body(ch, carry):
        offe = wid * (NCH * ECH) + ch * ECH
        pltpu.sync_copy(srcf.at[pl.ds(offe, ECH)], src_v)
        pltpu.sync_copy(dstgf.at[pl.ds(offe, ECH)], dstg_v)

        def grp_body(q, qcarry):
            src16 = src_v[pl.ds(q * 16, 16)]
            dstg16 = dstg_v[pl.ds(q * 16, 16)]
            ssl = src16 * 8
            dsl = dstg16 * 8
            for h in range(HEADS):
                als = plsc.load_gather(aa_v, [ssl + h])
                ald = plsc.load_gather(aa_v, [dsl + (HEADS + h)])
                e = als + ald
                e = jnp.where(e > 0, e, e * 0.2)
                kh = plsc.load_gather(kv_v, [_full16(h)])
                w16 = jnp.exp(e - kh)
                plsc.store_scatter(
                    wbuf_v, [(q * 16 + lanes0) * HEADS + h], w16)
            return qcarry

        lax.fori_loop(0, ECH // 16, grp_body, 0)
        pltpu.sync_copy(wbuf_v, wout.at[pl.ds(offe * HEADS, ECH * HEADS)])
        return carry

    lax.fori_loop(0, NCH, chunk_body, 0)


def _w_phase(aa, srcf, dstgf, kvec):
    f = pl.kernel(
        _w_body,
        out_type=[jax.ShapeDtypeStruct((NE * HEADS,), jnp.float32)],
        mesh=plsc.VectorSubcoreMesh(num_cores=2, num_subcores=NSUB, **_MESH),
        compiler_params=pltpu.CompilerParams(needs_layout_passes=False),
        scratch_types=[
            pltpu.VMEM((NP * 8,), jnp.float32),
            pltpu.VMEM((ECH,), jnp.int32),
            pltpu.VMEM((ECH,), jnp.int32),
            pltpu.VMEM((ECH * HEADS,), jnp.float32),
            pltpu.VMEM((128,), jnp.float32),
        ],
    )
    (w,) = f(aa.reshape(NP * 8), srcf, dstgf, kvec)
    return w


def _agg_body(hp, wall, srcs3, dstl2, zacc, zden,
              agg, denp,
              srcs_v, dstl_v, wrng_v, rows_v, acc_v, den_v, sem):
    c = lax.axis_index("c")
    s = lax.axis_index("s")
    wid = c * NSUB + s
    lanes0 = lax.iota(jnp.int32, 16)

    def range_body(p, carry):
        rid = p * NW + wid
        pltpu.sync_copy(srcs3.at[rid], srcs_v)
        pltpu.sync_copy(dstl2.at[rid], dstl_v)
        pltpu.sync_copy(wall.at[rid], wrng_v)
        pltpu.sync_copy(zacc, acc_v)
        pltpu.sync_copy(zden, den_v)

        def blk_body(b, bcarry):
            pltpu.async_copy(hp.at[srcs_v.at[b]], rows_v, sem).wait()

            for jj in range(G // 16):
                dst16 = dstl_v[pl.ds(b * G + jj * 16, 16)]
                widx = (b * G + jj * 16 + lanes0) * HEADS
                for h in range(HEADS):
                    wv = plsc.load_gather(wrng_v, [widx + h])
                    plsc.addupdate_scatter(den_v, [dst16 * HEADS + h], wv)

            def edge_scale(i, ecarry):
                dstb = plsc.load_gather(dstl_v, [_full16(b * G + i)])
                rowbase = dstb * HID
                for h in range(HEADS):
                    wb = plsc.load_gather(
                        wrng_v, [_full16((b * G + i) * HEADS + h)])
                    for j in range(C // 16):
                        col = h * C + j * 16
                        seg = rows_v[i, pl.ds(col, 16)]
                        plsc.addupdate_scatter(
                            acc_v, [rowbase + col + lanes0], seg * wb)
                return ecarry

            lax.fori_loop(0, G, edge_scale, 0)
            return bcarry

        lax.fori_loop(0, NBLK, blk_body, 0)

        pltpu.sync_copy(acc_v.at[pl.ds(0, R * HID)],
                        agg.at[pl.ds(rid * R * HID, R * HID)])
        pltpu.sync_copy(den_v.at[pl.ds(0, 384)], denp.at[rid])
        return carry

    lax.fori_loop(0, PHASES, range_body, 0)


def _agg_phase(hp, wall, srcs3, dstl2, zacc, zden):
    f = pl.kernel(
        _agg_body,
        out_type=[
            jax.ShapeDtypeStruct((NP * HID,), jnp.float32),
            jax.ShapeDtypeStruct((NRNG, 384), jnp.float32),
        ],
        mesh=plsc.VectorSubcoreMesh(num_cores=2, num_subcores=NSUB, **_MESH),
        compiler_params=pltpu.CompilerParams(needs_layout_passes=False),
        scratch_types=[
            pltpu.VMEM((NBLK, G), jnp.int32),
            pltpu.VMEM((RCAP,), jnp.int32),
            pltpu.VMEM((RCAP * HEADS,), jnp.float32),
            pltpu.VMEM((G, HID), jnp.float32),
            pltpu.VMEM(((R + 1) * HID,), jnp.float32),
            pltpu.VMEM((384,), jnp.float32),
            pltpu.SemaphoreType.DMA,
        ],
    )
    return f(hp, wall.reshape(NRNG, RCAP * HEADS), srcs3, dstl2, zacc, zden)


def _prep_edges(src, dst):
    order = jnp.argsort(dst)
    src_s = src[order]
    dst_s = dst[order]
    bounds = jnp.searchsorted(dst_s, jnp.arange(NRNG + 1) * R).astype(jnp.int32)
    src_pad = jnp.concatenate([src_s, jnp.zeros((RCAP,), jnp.int32)])
    dst_pad = jnp.concatenate([dst_s, jnp.zeros((RCAP,), jnp.int32)])
    ar = jnp.arange(RCAP, dtype=jnp.int32)
    idx = bounds[:NRNG, None] + ar[None, :]
    valid = idx < bounds[1:, None]
    srcs = jnp.where(valid, src_pad[idx], 0)
    dstg = jnp.where(valid, dst_pad[idx], 0)
    rid = jnp.arange(NRNG, dtype=jnp.int32)
    dstl = jnp.where(valid, dst_pad[idx] - rid[:, None] * R, R)
    srcs3 = srcs.reshape(NRNG, NBLK, G)
    srcf = srcs.reshape(NE)
    dstgf = dstg.reshape(NE)
    dstl2 = dstl.reshape(NRNG, RCAP)
    return srcs3, srcf, dstgf, dstl2


def kernel(x, edge_index, params):
    p = params
    src = edge_index[0]
    dst = edge_index[1]

    srcs3, srcf, dstgf, dstl2 = _prep_edges(src, dst)
    zacc = jnp.zeros(((R + 1) * HID,), jnp.float32)
    zden = jnp.zeros((384,), jnp.float32)

    xp = jnp.zeros((NP, F_IN), jnp.float32).at[:N].set(x)

    h = _dense(xp, p['W1'], _fuse_bn(p['b1'], p['g1'], p['be1']), True)
    h = _dense(h, p['W2'], _fuse_bn(p['b2'], p['g2'], p['be2']), True)

    for layer in (1, 2):
        W = p[f'Wg{layer}']
        a_s = p[f'as{layer}']
        a_d = p[f'ad{layer}']
        bias = p[f'bg{layer}']
        gb = p[f'gb{layer}']
        bb = p[f'bb{layer}']

        As = jnp.zeros((HID, HEADS), jnp.float32)
        Ad = jnp.zeros((HID, HEADS), jnp.float32)
        for hh in range(HEADS):
            As = As.at[hh * C:(hh + 1) * C, hh].set(a_s[hh])
            Ad = Ad.at[hh * C:(hh + 1) * C, hh].set(a_d[hh])
        Waa = jnp.concatenate([jnp.dot(W, As), jnp.dot(W, Ad)], axis=1)

        ident = jnp.stack([jnp.ones((HID,), jnp.float32),
                           jnp.zeros((HID,), jnp.float32)])
        hp = _dense(h, W, ident, False)
        ident8 = jnp.stack([jnp.ones((8,), jnp.float32),
                            jnp.zeros((8,), jnp.float32)])
        aa = _dense(h, Waa, ident8, False)

        als = aa[:N, :HEADS]
        ald = aa[:N, HEADS:]
        K_h = jnp.maximum(jnp.max(als, axis=0) + jnp.max(ald, axis=0), 0.0)
        kvec = jnp.zeros((128,), jnp.float32).at[:HEADS].set(K_h)

        wall = _w_phase(aa, srcf, dstgf, kvec)
        aggf, denp = _agg_phase(hp, wall, srcs3, dstl2, zacc, zden)

        agg = aggf.reshape(NP, HID)[:N]
        den = denp[:, :R * HEADS].reshape(NP, HEADS)[:N]
        aggn = agg.reshape(N, HEADS, C) / (den[:, :, None] + 1e-16)
        out = aggn.reshape(N, HID) + bias

        scale = gb * _INV_SQRT
        y = out * scale[None, :] + bb[None, :]
        y = jnp.where(y > 0, y, 0.01 * y)
        h = jnp.zeros((NP, HID), jnp.float32).at[:N].set(y)

    h = _dense(h, p['L1W'], _fuse_bn(p['L1b'], jnp.ones((HID,)), jnp.zeros((HID,))), True)
    h = _dense(h, p['L2W'], _fuse_bn(p['L2b'], jnp.ones((HID // 2,)), jnp.zeros((HID // 2,))), True)
    L3Wp = jnp.zeros((HID // 2, 128), jnp.float32).at[:, :1].set(p['L3W'])
    ss = jnp.stack([jnp.ones((128,), jnp.float32),
                    jnp.zeros((128,), jnp.float32).at[0].set(p['L3b'][0])])
    out = _dense(h, L3Wp, ss, False)
    return out[:N, :1]
